# baseline (speedup 1.0000x reference)
import functools

import numpy as np
import jax
import jax.numpy as jnp
from jax import lax
from jax.experimental import pallas as pl
from jax.experimental.pallas import tpu as pltpu
from jax.experimental.pallas import tpu_sc as plsc

_NE = 8
_D = 1024
_T = 4096
_K = 2
_NP = _T * _K
_B = 256
_P = _NP + _NE * _B
_NB = _P // _B
_NW = 32
_GC = 64


def _build_perms():
    p = np.arange(_D)
    b = (p >= _D // 2).astype(np.int64)
    m = p - (_D // 2) * b
    kperm = 32 * (m // 16) + 2 * (m % 16) + b
    rowperm = np.concatenate([2 * kperm, 2 * kperm + 1])
    return kperm, rowperm


_KPERM, _ROWPERM = _build_perms()


def _decode_fp4(nib, scale):
    mag = nib & 7
    mf = mag.astype(jnp.float32)
    dec = jnp.where(mag < 5, mf * 0.5, mf - 2.0)
    dec = jnp.where(mag == 7, 6.0, dec)
    sgn = jnp.where((nib & 8) == 0, 1.0, -1.0)
    return dec * sgn * scale


def _decode_body(gub, gus, dnb, dns, gw_out, dw_out):
    def dec(bref, sref, oref):
        bv = bref[0].astype(jnp.int32)
        sv = sref[0].astype(jnp.int32)
        scale = jax.lax.bitcast_convert_type(sv << 23, jnp.float32)
        oref[0, :, : _D // 2] = _decode_fp4(bv & 15, scale).astype(jnp.bfloat16)
        oref[0, :, _D // 2:] = _decode_fp4(bv >> 4, scale).astype(jnp.bfloat16)

    dec(gub, gus, gw_out)
    dec(dnb, dns, dw_out)


def _decode_weights(gub, gus, dnb, dns):
    return pl.pallas_call(
        _decode_body,
        grid=(_NE,),
        in_specs=[
            pl.BlockSpec((1, 2 * _D, _D // 2), lambda e: (e, 0, 0)),
            pl.BlockSpec((1, 2 * _D, _D // 2), lambda e: (e, 0, 0)),
            pl.BlockSpec((1, _D, _D // 2), lambda e: (e, 0, 0)),
            pl.BlockSpec((1, _D, _D // 2), lambda e: (e, 0, 0)),
        ],
        out_specs=[
            pl.BlockSpec((1, 2 * _D, _D), lambda e: (e, 0, 0)),
            pl.BlockSpec((1, _D, _D), lambda e: (e, 0, 0)),
        ],
        out_shape=[
            jax.ShapeDtypeStruct((_NE, 2 * _D, _D), jnp.bfloat16),
            jax.ShapeDtypeStruct((_NE, _D, _D), jnp.bfloat16),
        ],
    )(gub, gus, dnb, dns)


def _bf16_rows_to_i32(x):
    v = x.shape[0]
    return jax.lax.bitcast_convert_type(
        x.reshape(v, _D // 2, 2), jnp.int32).reshape(v, 4, 128)


def _i32_rows_to_bf16(x):
    v = x.shape[0]
    return jax.lax.bitcast_convert_type(
        x.reshape(v, _D // 2), jnp.bfloat16).reshape(v, _D)


def _sc_gather_rows(table3, rid, n_rows):
    per_w = n_rows // _NW
    n_chunks = per_w // _GC
    mesh = plsc.VectorSubcoreMesh(core_axis_name="c", subcore_axis_name="s")

    @functools.partial(
        pl.kernel,
        mesh=mesh,
        out_type=jax.ShapeDtypeStruct((n_rows, 4, 128), jnp.int32),
        scratch_types=[
            pltpu.VMEM((_GC,), jnp.int32),
            pltpu.VMEM((_GC, 4, 128), jnp.int32),
            pltpu.SemaphoreType.DMA,
        ],
    )
    def k(table_hbm, rid_hbm, out_hbm, idx_v, rows_v, sem):
        wid = lax.axis_index("s") * 2 + lax.axis_index("c")
        base = wid * per_w
        for c in range(n_chunks):
            b = base + c * _GC
            pltpu.sync_copy(rid_hbm.at[pl.ds(b, _GC)], idx_v)
            pltpu.async_copy(table_hbm.at[idx_v], rows_v, sem).wait()
            pltpu.sync_copy(rows_v, out_hbm.at[pl.ds(b, _GC)])

    return k(table3, rid)


def _ffn_body(be_ref, lim_ref, x_ref, gw_ref, b1_ref, dw_ref, b2_ref,
              wv_ref, out_ref):
    limv = lim_ref[0, 0]
    x = x_ref[...]
    gu = jax.lax.dot_general(
        x, gw_ref[0], (((1,), (1,)), ((), ())),
        preferred_element_type=jnp.float32)
    gu = gu + b1_ref[0]
    g = jnp.minimum(gu[:, :_D], limv)
    l = jnp.clip(gu[:, _D:], -limv, limv)
    act = (g * jax.nn.sigmoid(1.702 * g) * (l + 1.0)).astype(jnp.bfloat16)
    y = jax.lax.dot_general(
        act, dw_ref[0], (((1,), (1,)), ((), ())),
        preferred_element_type=jnp.float32)
    y = y + b2_ref[0]
    w = wv_ref[0, 0]
    out_ref[...] = (y * w[:, None]).astype(jnp.bfloat16)


def _ffn_grouped(block_expert, lim, xg, gw, b1, dw, b2, wv):
    grid_spec = pltpu.PrefetchScalarGridSpec(
        num_scalar_prefetch=1,
        grid=(_NB,),
        in_specs=[
            pl.BlockSpec(memory_space=pltpu.SMEM),
            pl.BlockSpec((_B, _D), lambda i, be: (i, 0)),
            pl.BlockSpec((1, 2 * _D, _D), lambda i, be: (be[i], 0, 0)),
            pl.BlockSpec((1, 1, 2 * _D), lambda i, be: (be[i], 0, 0)),
            pl.BlockSpec((1, _D, _D), lambda i, be: (be[i], 0, 0)),
            pl.BlockSpec((1, 1, _D), lambda i, be: (be[i], 0, 0)),
            pl.BlockSpec((1, 1, _B), lambda i, be: (i, 0, 0)),
        ],
        out_specs=pl.BlockSpec((_B, _D), lambda i, be: (i, 0)),
    )
    return pl.pallas_call(
        _ffn_body,
        grid_spec=grid_spec,
        out_shape=jax.ShapeDtypeStruct((_P, _D), jnp.bfloat16),
    )(block_expert, lim, xg, gw, b1, dw, b2, wv)


def _combine_body(a_ref, b_ref, out_ref):
    out_ref[...] = a_ref[...].astype(jnp.float32) + b_ref[...].astype(jnp.float32)


def _combine_add(g):
    nb = 8
    rb = _T // nb
    return pl.pallas_call(
        _combine_body,
        grid=(nb,),
        in_specs=[
            pl.BlockSpec((rb, _D), lambda i: (i, 0)),
            pl.BlockSpec((rb, _D), lambda i: (i + nb, 0)),
        ],
        out_specs=pl.BlockSpec((rb, _D), lambda i: (i, 0)),
        out_shape=jax.ShapeDtypeStruct((_T, _D), jnp.float32),
    )(g, g)


def kernel(hidden_states, router_indices, routing_weights, swiglu_limit,
           gate_up_proj_blocks, gate_up_proj_scales, gate_up_proj_bias,
           down_proj_blocks, down_proj_scales, down_proj_bias):
    flat = hidden_states.reshape(-1, _D)
    xq = flat[:, _KPERM].astype(jnp.bfloat16)

    gub = gate_up_proj_blocks.reshape(_NE, 2 * _D, _D // 2)[:, _ROWPERM]
    gus = jnp.repeat(gate_up_proj_scales[:, _ROWPERM], 16, axis=-1)
    dnb = down_proj_blocks.reshape(_NE, _D, _D // 2)
    dns = jnp.repeat(down_proj_scales, 16, axis=-1)
    b1p = gate_up_proj_bias[:, _ROWPERM].reshape(_NE, 1, 2 * _D)
    b2r = down_proj_bias.reshape(_NE, 1, _D)

    e_flat = router_indices.reshape(-1).astype(jnp.int32)
    sort_idx = jnp.argsort(e_flat)
    e_sorted = jnp.take(e_flat, sort_idx)
    counts = (e_flat[None, :] == jnp.arange(_NE, dtype=jnp.int32)[:, None]
              ).astype(jnp.int32).sum(axis=1)
    offs = jnp.concatenate([jnp.zeros((1,), jnp.int32),
                            jnp.cumsum(counts)[:-1]])
    padded = ((counts + _B - 1) // _B) * _B
    pad_ends = jnp.cumsum(padded)
    pad_offs = pad_ends - padded
    rank = jnp.arange(_NP, dtype=jnp.int32) - jnp.take(offs, e_sorted)
    dest = jnp.take(pad_offs, e_sorted) + rank

    tok = sort_idx // _K
    row_id = jnp.zeros((_P,), jnp.int32).at[dest].set(tok)
    pair_w = routing_weights[tok, e_sorted]
    wv = jnp.zeros((_P,), jnp.float32).at[dest].set(pair_w)

    block_expert = jnp.clip(
        jnp.searchsorted(pad_ends, jnp.arange(_NB, dtype=jnp.int32) * _B,
                         side="right"), 0, _NE - 1).astype(jnp.int32)

    inv = jnp.zeros((_NP,), jnp.int32).at[sort_idx].set(dest)
    p01 = jnp.concatenate([inv[0::2], inv[1::2]])

    lim = jnp.full((1, 1), swiglu_limit, jnp.float32)

    gw, dw = _decode_weights(gub, gus, dnb, dns)
    xg3 = _sc_gather_rows(_bf16_rows_to_i32(xq), row_id, _P)
    r = _ffn_grouped(block_expert, lim, _i32_rows_to_bf16(xg3), gw, b1p, dw,
                     b2r, wv.reshape(_NB, 1, _B))
    g3 = _sc_gather_rows(_bf16_rows_to_i32(r), p01, _K * _T)
    out = _combine_add(_i32_rows_to_bf16(g3))
    return out.reshape(*hidden_states.shape[:-1], _D).astype(hidden_states.dtype)


# speedup vs baseline: 1.0230x; 1.0230x over previous
import functools

import numpy as np
import jax
import jax.numpy as jnp
from jax import lax
from jax.experimental import pallas as pl
from jax.experimental.pallas import tpu as pltpu
from jax.experimental.pallas import tpu_sc as plsc

_NE = 8
_D = 1024
_T = 4096
_K = 2
_NP = _T * _K
_B = 256
_P = _NP + _NE * _B
_NB = _P // _B
_NW = 32


def _decode_fp4(nib, scale):
    mag = nib & 7
    mf = mag.astype(jnp.float32)
    dec = jnp.where(mag < 5, mf * 0.5, mf - 2.0)
    dec = jnp.where(mag == 7, 6.0, dec)
    sgn = jnp.where((nib & 8) == 0, 1.0, -1.0)
    return dec * sgn * scale


def _decode_body(gub, gus, dnb, dns, gw_out, dw_out):
    def dec(bref, sref, oref):
        bv = bref[0].astype(jnp.int32)
        sv = sref[0].astype(jnp.int32)
        scale = jax.lax.bitcast_convert_type(sv << 23, jnp.float32)
        oref[0, :, : _D // 2] = _decode_fp4(bv & 15, scale).astype(jnp.bfloat16)
        oref[0, :, _D // 2:] = _decode_fp4(bv >> 4, scale).astype(jnp.bfloat16)

    dec(gub, gus, gw_out)
    dec(dnb, dns, dw_out)


def _decode_weights(gub, gus, dnb, dns):
    return pl.pallas_call(
        _decode_body,
        grid=(_NE,),
        in_specs=[
            pl.BlockSpec((1, 2 * _D, _D // 2), lambda e: (e, 0, 0)),
            pl.BlockSpec((1, 2 * _D, _D // 2), lambda e: (e, 0, 0)),
            pl.BlockSpec((1, _D, _D // 2), lambda e: (e, 0, 0)),
            pl.BlockSpec((1, _D, _D // 2), lambda e: (e, 0, 0)),
        ],
        out_specs=[
            pl.BlockSpec((1, 2 * _D, _D), lambda e: (e, 0, 0)),
            pl.BlockSpec((1, _D, _D), lambda e: (e, 0, 0)),
        ],
        out_shape=[
            jax.ShapeDtypeStruct((_NE, 2 * _D, _D), jnp.bfloat16),
            jax.ShapeDtypeStruct((_NE, _D, _D), jnp.bfloat16),
        ],
    )(gub, gus, dnb, dns)


def _bf16_rows_to_i32(x):
    v = x.shape[0]
    return jax.lax.bitcast_convert_type(
        x.reshape(v, _D // 2, 2), jnp.int32).reshape(v, 4, 128)


def _i32_rows_to_bf16(x):
    v = x.shape[0]
    return jax.lax.bitcast_convert_type(
        x.reshape(v, _D // 2), jnp.bfloat16).reshape(v, _D)


def _sc_dispatch_rows(xq_i32, p0, p1):
    per_w = _T // _NW
    mesh = plsc.VectorSubcoreMesh(core_axis_name="c", subcore_axis_name="s")

    @functools.partial(
        pl.kernel,
        mesh=mesh,
        out_type=jax.ShapeDtypeStruct((_P, 4, 128), jnp.int32),
        scratch_types=[
            pltpu.VMEM((per_w,), jnp.int32),
            pltpu.VMEM((per_w,), jnp.int32),
            pltpu.VMEM((per_w, 4, 128), jnp.int32),
            pltpu.SemaphoreType.DMA,
            pltpu.SemaphoreType.DMA,
        ],
    )
    def k(x_hbm, p0_hbm, p1_hbm, out_hbm, i0_v, i1_v, rows_v, s0, s1):
        wid = lax.axis_index("s") * 2 + lax.axis_index("c")
        base = wid * per_w
        pltpu.sync_copy(p0_hbm.at[pl.ds(base, per_w)], i0_v)
        pltpu.sync_copy(p1_hbm.at[pl.ds(base, per_w)], i1_v)
        pltpu.sync_copy(x_hbm.at[pl.ds(base, per_w)], rows_v)
        h0 = pltpu.async_copy(rows_v, out_hbm.at[i0_v], s0)
        h1 = pltpu.async_copy(rows_v, out_hbm.at[i1_v], s1)
        h0.wait()
        h1.wait()

    return k(xq_i32, p0, p1)


def _sc_gather_rows(table3, rid, n_rows):
    per_w = n_rows // _NW
    gc = 64
    n_chunks = per_w // gc
    mesh = plsc.VectorSubcoreMesh(core_axis_name="c", subcore_axis_name="s")

    @functools.partial(
        pl.kernel,
        mesh=mesh,
        out_type=jax.ShapeDtypeStruct((n_rows, 4, 128), jnp.int32),
        scratch_types=[
            pltpu.VMEM((per_w,), jnp.int32),
            pltpu.VMEM((gc, 4, 128), jnp.int32),
            pltpu.VMEM((gc, 4, 128), jnp.int32),
            pltpu.SemaphoreType.DMA,
            pltpu.SemaphoreType.DMA,
        ],
    )
    def k(table_hbm, rid_hbm, out_hbm, idx_v, b0, b1, s0, s1):
        wid = lax.axis_index("s") * 2 + lax.axis_index("c")
        base = wid * per_w
        pltpu.sync_copy(rid_hbm.at[pl.ds(base, per_w)], idx_v)
        bufs = (b0, b1)
        sems = (s0, s1)
        hs = [None] * n_chunks
        hs[0] = pltpu.async_copy(
            table_hbm.at[idx_v.at[pl.ds(0, gc)]], bufs[0], sems[0])
        for c in range(n_chunks):
            if c + 1 < n_chunks:
                hs[c + 1] = pltpu.async_copy(
                    table_hbm.at[idx_v.at[pl.ds((c + 1) * gc, gc)]],
                    bufs[(c + 1) % 2], sems[(c + 1) % 2])
            hs[c].wait()
            pltpu.sync_copy(bufs[c % 2], out_hbm.at[pl.ds(base + c * gc, gc)])

    return k(table3, rid)


def _ffn_body(be_ref, lim_ref, x_ref, gw_ref, b1_ref, dw_ref, b2_ref, out_ref):
    limv = lim_ref[0, 0]
    x = x_ref[...]
    gu = jax.lax.dot_general(
        x, gw_ref[0], (((1,), (1,)), ((), ())),
        preferred_element_type=jnp.float32)
    gu = gu + b1_ref[0]
    g = jnp.minimum(gu[:, :_D], limv)
    l = jnp.clip(gu[:, _D:], -limv, limv)
    act = (g * jax.nn.sigmoid(1.702 * g) * (l + 1.0)).astype(jnp.bfloat16)
    y = jax.lax.dot_general(
        act, dw_ref[0], (((1,), (1,)), ((), ())),
        preferred_element_type=jnp.float32)
    y = y + b2_ref[0]
    out_ref[...] = y.astype(jnp.bfloat16)


def _ffn_grouped(block_expert, lim, xg, gw, b1, dw, b2):
    grid_spec = pltpu.PrefetchScalarGridSpec(
        num_scalar_prefetch=1,
        grid=(_NB,),
        in_specs=[
            pl.BlockSpec(memory_space=pltpu.SMEM),
            pl.BlockSpec((_B, _D), lambda i, be: (i, 0)),
            pl.BlockSpec((1, 2 * _D, _D), lambda i, be: (be[i], 0, 0)),
            pl.BlockSpec((1, 1, 2 * _D), lambda i, be: (be[i], 0, 0)),
            pl.BlockSpec((1, _D, _D), lambda i, be: (be[i], 0, 0)),
            pl.BlockSpec((1, 1, _D), lambda i, be: (be[i], 0, 0)),
        ],
        out_specs=pl.BlockSpec((_B, _D), lambda i, be: (i, 0)),
    )
    return pl.pallas_call(
        _ffn_body,
        grid_spec=grid_spec,
        out_shape=jax.ShapeDtypeStruct((_P, _D), jnp.bfloat16),
    )(block_expert, lim, xg, gw, b1, dw, b2)


def _combine_body(a_ref, b_ref, rw_ref, ri_ref, out_ref):
    ri = ri_ref[...]
    rw = rw_ref[...]
    eids = jax.lax.broadcasted_iota(jnp.int32, (ri.shape[0], _NE), 1)
    w0 = jnp.sum(jnp.where(eids == ri[:, 0:1], rw, 0.0), axis=1)
    w1 = jnp.sum(jnp.where(eids == ri[:, 1:2], rw, 0.0), axis=1)
    a = a_ref[...].astype(jnp.float32)
    b = b_ref[...].astype(jnp.float32)
    out_ref[...] = w0[:, None] * a + w1[:, None] * b


def _combine_add(g, rw, ri):
    nb = 8
    rb = _T // nb
    return pl.pallas_call(
        _combine_body,
        grid=(nb,),
        in_specs=[
            pl.BlockSpec((rb, _D), lambda i: (i, 0)),
            pl.BlockSpec((rb, _D), lambda i: (i + nb, 0)),
            pl.BlockSpec((rb, _NE), lambda i: (i, 0)),
            pl.BlockSpec((rb, _K), lambda i: (i, 0)),
        ],
        out_specs=pl.BlockSpec((rb, _D), lambda i: (i, 0)),
        out_shape=jax.ShapeDtypeStruct((_T, _D), jnp.float32),
    )(g, g, rw, ri)


def kernel(hidden_states, router_indices, routing_weights, swiglu_limit,
           gate_up_proj_blocks, gate_up_proj_scales, gate_up_proj_bias,
           down_proj_blocks, down_proj_scales, down_proj_bias):
    flat = hidden_states.reshape(-1, _D)
    xq = flat.reshape(_T, 32, 16, 2).transpose(0, 3, 1, 2).reshape(_T, _D)
    xq = xq.astype(jnp.bfloat16)

    def rowperm(a, tail):
        return a.reshape(_NE, 32, 16, 2, 2, *tail).transpose(
            0, 4, 3, 1, 2, *range(5, 5 + len(tail))).reshape(
            _NE, 2 * _D, *tail)

    gub = rowperm(gate_up_proj_blocks.reshape(_NE, 2 * _D, _D // 2), (_D // 2,))
    gus = jnp.repeat(rowperm(gate_up_proj_scales, (32,)), 16, axis=-1)
    dnb = down_proj_blocks.reshape(_NE, _D, _D // 2)
    dns = jnp.repeat(down_proj_scales, 16, axis=-1)
    b1p = rowperm(gate_up_proj_bias, ()).reshape(_NE, 1, 2 * _D)
    b2r = down_proj_bias.reshape(_NE, 1, _D)

    ri = router_indices.astype(jnp.int32)
    e_flat = ri.reshape(-1)
    onehot = (e_flat[:, None] == jnp.arange(_NE, dtype=jnp.int32)[None, :]
              ).astype(jnp.int32)
    ranks_inc = jnp.cumsum(onehot, axis=0)
    counts = ranks_inc[-1]
    padded = ((counts + _B - 1) // _B) * _B
    pad_ends = jnp.cumsum(padded)
    pad_offs = pad_ends - padded
    rank = jnp.sum((ranks_inc - onehot) * onehot, axis=1)
    dest = jnp.sum(pad_offs[None, :] * onehot, axis=1) + rank
    d2 = dest.reshape(_T, _K)
    p0 = d2[:, 0].astype(jnp.int32)
    p1 = d2[:, 1].astype(jnp.int32)

    block_expert = jnp.sum(
        (jnp.arange(_NB, dtype=jnp.int32)[:, None] * _B >= pad_ends[None, :]
         ).astype(jnp.int32), axis=1)
    block_expert = jnp.minimum(block_expert, _NE - 1).astype(jnp.int32)

    lim = jnp.full((1, 1), swiglu_limit, jnp.float32)

    gw, dw = _decode_weights(gub, gus, dnb, dns)
    xg3 = _sc_dispatch_rows(_bf16_rows_to_i32(xq), p0, p1)
    r = _ffn_grouped(block_expert, lim, _i32_rows_to_bf16(xg3), gw, b1p, dw,
                     b2r)
    p01 = jnp.concatenate([p0, p1])
    g3 = _sc_gather_rows(_bf16_rows_to_i32(r), p01, _K * _T)
    out = _combine_add(_i32_rows_to_bf16(g3), routing_weights, ri)
    return out.reshape(*hidden_states.shape[:-1], _D).astype(hidden_states.dtype)


# speedup vs baseline: 2.0233x; 1.9778x over previous
import functools

import numpy as np
import jax
import jax.numpy as jnp
from jax import lax
from jax.experimental import pallas as pl
from jax.experimental.pallas import tpu as pltpu
from jax.experimental.pallas import tpu_sc as plsc

_NE = 8
_D = 1024
_T = 4096
_K = 2
_NP = _T * _K
_B = 256
_P = _NP + _NE * _B
_NB = _P // _B
_NW = 32


def _consts():
    p = np.arange(_D)
    b = (p >= _D // 2).astype(np.int64)
    m = p - (_D // 2) * b
    kperm = 32 * (m // 16) + 2 * (m % 16) + b
    mm = np.zeros((_D, _D), np.float32)
    mm[np.arange(_D), kperm] = 1.0
    m_lo = mm[: _D // 2]
    m_hi = mm[_D // 2:]
    s = np.zeros((32, _D // 2), np.float32)
    s[np.arange(_D // 2) // 16, np.arange(_D // 2)] = 1.0
    rowperm = np.concatenate([2 * kperm, 2 * kperm + 1])
    return m_lo, m_hi, s, rowperm


_MLO, _MHI, _SSEL, _ROWPERM = _consts()


def _decode_fp4(nib, scale):
    mag = nib & 7
    mf = mag.astype(jnp.float32)
    dec = jnp.where(mag < 5, mf * 0.5, mf - 2.0)
    dec = jnp.where(mag == 7, 6.0, dec)
    sgn = jnp.where((nib & 8) == 0, 1.0, -1.0)
    return dec * sgn * scale


def _halves(bref, sref, ssel):
    bv = bref[0].astype(jnp.int32)
    sv = sref[0].astype(jnp.int32)
    sc32 = jax.lax.bitcast_convert_type(sv << 23, jnp.float32)
    scale = jax.lax.dot_general(
        sc32, ssel, (((1,), (0,)), ((), ())),
        preferred_element_type=jnp.float32)
    lo = _decode_fp4(bv & 15, scale).astype(jnp.bfloat16)
    hi = _decode_fp4(bv >> 4, scale).astype(jnp.bfloat16)
    return lo, hi


def _decode_body(gub, gus, dnb, dns, mlo_ref, mhi_ref, ssel_ref, gw_out,
                 dw_out):
    ssel = ssel_ref[...]
    lo, hi = _halves(gub, gus, ssel)
    nat = jax.lax.dot_general(
        lo, mlo_ref[...], (((1,), (0,)), ((), ())),
        preferred_element_type=jnp.float32)
    nat = nat + jax.lax.dot_general(
        hi, mhi_ref[...], (((1,), (0,)), ((), ())),
        preferred_element_type=jnp.float32)
    gw_out[0] = nat.astype(jnp.bfloat16)
    lo, hi = _halves(dnb, dns, ssel)
    dw_out[0, :, : _D // 2] = lo
    dw_out[0, :, _D // 2:] = hi


def _decode_weights(gub, gus, dnb, dns):
    return pl.pallas_call(
        _decode_body,
        grid=(_NE,),
        in_specs=[
            pl.BlockSpec((1, 2 * _D, _D // 2), lambda e: (e, 0, 0)),
            pl.BlockSpec((1, 2 * _D, 32), lambda e: (e, 0, 0)),
            pl.BlockSpec((1, _D, _D // 2), lambda e: (e, 0, 0)),
            pl.BlockSpec((1, _D, 32), lambda e: (e, 0, 0)),
            pl.BlockSpec((_D // 2, _D), lambda e: (0, 0)),
            pl.BlockSpec((_D // 2, _D), lambda e: (0, 0)),
            pl.BlockSpec((32, _D // 2), lambda e: (0, 0)),
        ],
        out_specs=[
            pl.BlockSpec((1, 2 * _D, _D), lambda e: (e, 0, 0)),
            pl.BlockSpec((1, _D, _D), lambda e: (e, 0, 0)),
        ],
        out_shape=[
            jax.ShapeDtypeStruct((_NE, 2 * _D, _D), jnp.bfloat16),
            jax.ShapeDtypeStruct((_NE, _D, _D), jnp.bfloat16),
        ],
    )(gub, gus, dnb, dns, jnp.asarray(_MLO, jnp.bfloat16),
      jnp.asarray(_MHI, jnp.bfloat16), jnp.asarray(_SSEL, jnp.float32))


_DC = 32
_DNC = (_T // _NW) // _DC


def _sc_dispatch_rows(x3, p0r, p1r):
    per_w = _T // _NW
    mesh = plsc.VectorSubcoreMesh(core_axis_name="c", subcore_axis_name="s")

    @functools.partial(
        pl.kernel,
        mesh=mesh,
        out_type=jax.ShapeDtypeStruct((_P, 8, 128), jnp.float32),
        scratch_types=[
            pltpu.VMEM((_DNC, _DC), jnp.int32),
            pltpu.VMEM((_DNC, _DC), jnp.int32),
            pltpu.VMEM((_DC, 8, 128), jnp.float32),
            pltpu.VMEM((_DC, 8, 128), jnp.float32),
            pltpu.SemaphoreType.DMA,
            pltpu.SemaphoreType.DMA,
        ],
    )
    def k(x_hbm, p0_hbm, p1_hbm, out_hbm, i0_v, i1_v, r0_v, r1_v, s0, s1):
        wid = lax.axis_index("s") * 2 + lax.axis_index("c")
        base = wid * per_w
        pltpu.sync_copy(p0_hbm.at[wid], i0_v)
        pltpu.sync_copy(p1_hbm.at[wid], i1_v)
        bufs = (r0_v, r1_v)
        sems = (s0, s1)
        hs = [None] * _DNC
        for c in range(_DNC):
            if c >= 2:
                hs[c - 2][0].wait()
                hs[c - 2][1].wait()
            pltpu.sync_copy(x_hbm.at[pl.ds(base + c * _DC, _DC)], bufs[c % 2])
            h0 = pltpu.async_copy(bufs[c % 2], out_hbm.at[i0_v.at[c]],
                                  sems[c % 2])
            h1 = pltpu.async_copy(bufs[c % 2], out_hbm.at[i1_v.at[c]],
                                  sems[c % 2])
            hs[c] = (h0, h1)
        for c in range(max(_DNC - 2, 0), _DNC):
            hs[c][0].wait()
            hs[c][1].wait()

    return k(x3, p0r, p1r)


def _sc_gather_rows(table3, rid, n_rows):
    per_w = n_rows // _NW
    gc = 32
    n_chunks = per_w // gc
    mesh = plsc.VectorSubcoreMesh(core_axis_name="c", subcore_axis_name="s")

    @functools.partial(
        pl.kernel,
        mesh=mesh,
        out_type=jax.ShapeDtypeStruct((n_rows, 8, 128), jnp.float32),
        scratch_types=[
            pltpu.VMEM((per_w,), jnp.int32),
            pltpu.VMEM((gc, 8, 128), jnp.float32),
            pltpu.VMEM((gc, 8, 128), jnp.float32),
            pltpu.SemaphoreType.DMA,
            pltpu.SemaphoreType.DMA,
        ],
    )
    def k(table_hbm, rid_hbm, out_hbm, idx_v, b0, b1, s0, s1):
        wid = lax.axis_index("s") * 2 + lax.axis_index("c")
        base = wid * per_w
        pltpu.sync_copy(rid_hbm.at[pl.ds(base, per_w)], idx_v)
        bufs = (b0, b1)
        sems = (s0, s1)
        hs = [None] * n_chunks
        hs[0] = pltpu.async_copy(
            table_hbm.at[idx_v.at[pl.ds(0, gc)]], bufs[0], sems[0])
        for c in range(n_chunks):
            if c + 1 < n_chunks:
                hs[c + 1] = pltpu.async_copy(
                    table_hbm.at[idx_v.at[pl.ds((c + 1) * gc, gc)]],
                    bufs[(c + 1) % 2], sems[(c + 1) % 2])
            hs[c].wait()
            pltpu.sync_copy(bufs[c % 2], out_hbm.at[pl.ds(base + c * gc, gc)])

    return k(table3, rid)


def _ffn_body(be_ref, lim_ref, x_ref, gw_ref, b1_ref, dw_ref, b2_ref, out_ref):
    limv = lim_ref[0, 0]
    x = x_ref[...].astype(jnp.bfloat16)
    gu = jax.lax.dot_general(
        x, gw_ref[0], (((1,), (1,)), ((), ())),
        preferred_element_type=jnp.float32)
    gu = gu + b1_ref[0]
    g = jnp.minimum(gu[:, :_D], limv)
    l = jnp.clip(gu[:, _D:], -limv, limv)
    act = (g * jax.nn.sigmoid(1.702 * g) * (l + 1.0)).astype(jnp.bfloat16)
    y = jax.lax.dot_general(
        act, dw_ref[0], (((1,), (1,)), ((), ())),
        preferred_element_type=jnp.float32)
    y = y + b2_ref[0]
    out_ref[...] = y


def _ffn_grouped(block_expert, lim, xg, gw, b1, dw, b2):
    grid_spec = pltpu.PrefetchScalarGridSpec(
        num_scalar_prefetch=1,
        grid=(_NB,),
        in_specs=[
            pl.BlockSpec(memory_space=pltpu.SMEM),
            pl.BlockSpec((_B, _D), lambda i, be: (i, 0)),
            pl.BlockSpec((1, 2 * _D, _D), lambda i, be: (be[i], 0, 0)),
            pl.BlockSpec((1, 1, 2 * _D), lambda i, be: (be[i], 0, 0)),
            pl.BlockSpec((1, _D, _D), lambda i, be: (be[i], 0, 0)),
            pl.BlockSpec((1, 1, _D), lambda i, be: (be[i], 0, 0)),
        ],
        out_specs=pl.BlockSpec((_B, _D), lambda i, be: (i, 0)),
    )
    return pl.pallas_call(
        _ffn_body,
        grid_spec=grid_spec,
        out_shape=jax.ShapeDtypeStruct((_P, _D), jnp.float32),
    )(block_expert, lim, xg, gw, b1, dw, b2)


def _combine_body(a_ref, b_ref, rw_ref, ri_ref, out_ref):
    ri = ri_ref[...]
    rw = rw_ref[...]
    eids = jax.lax.broadcasted_iota(jnp.int32, (ri.shape[0], _NE), 1)
    w0 = jnp.sum(jnp.where(eids == ri[:, 0:1], rw, 0.0), axis=1)
    w1 = jnp.sum(jnp.where(eids == ri[:, 1:2], rw, 0.0), axis=1)
    out_ref[...] = w0[:, None] * a_ref[...] + w1[:, None] * b_ref[...]


def _combine_add(g, rw, ri):
    nb = 8
    rb = _T // nb
    return pl.pallas_call(
        _combine_body,
        grid=(nb,),
        in_specs=[
            pl.BlockSpec((rb, _D), lambda i: (i, 0)),
            pl.BlockSpec((rb, _D), lambda i: (i + nb, 0)),
            pl.BlockSpec((rb, _NE), lambda i: (i, 0)),
            pl.BlockSpec((rb, _K), lambda i: (i, 0)),
        ],
        out_specs=pl.BlockSpec((rb, _D), lambda i: (i, 0)),
        out_shape=jax.ShapeDtypeStruct((_T, _D), jnp.float32),
    )(g, g, rw, ri)


def kernel(hidden_states, router_indices, routing_weights, swiglu_limit,
           gate_up_proj_blocks, gate_up_proj_scales, gate_up_proj_bias,
           down_proj_blocks, down_proj_scales, down_proj_bias):
    flat = hidden_states.reshape(-1, _D)

    def rowperm(a, tail):
        return a.reshape(_NE, 32, 16, 2, 2, *tail).transpose(
            0, 4, 3, 1, 2, *range(5, 5 + len(tail))).reshape(
            _NE, 2 * _D, *tail)

    gub = rowperm(gate_up_proj_blocks.reshape(_NE, 2 * _D, _D // 2), (_D // 2,))
    gus = rowperm(gate_up_proj_scales, (32,))
    dnb = down_proj_blocks.reshape(_NE, _D, _D // 2)
    dns = down_proj_scales
    b1p = rowperm(gate_up_proj_bias, ()).reshape(_NE, 1, 2 * _D)
    b2r = down_proj_bias.reshape(_NE, 1, _D)

    ri = router_indices.astype(jnp.int32)
    e_flat = ri.reshape(-1)
    onehot = (e_flat[:, None] == jnp.arange(_NE, dtype=jnp.int32)[None, :]
              ).astype(jnp.int32)
    ranks_inc = jnp.cumsum(onehot, axis=0)
    counts = ranks_inc[-1]
    padded = ((counts + _B - 1) // _B) * _B
    pad_ends = jnp.cumsum(padded)
    pad_offs = pad_ends - padded
    rank = jnp.sum((ranks_inc - onehot) * onehot, axis=1)
    dest = jnp.sum(pad_offs[None, :] * onehot, axis=1) + rank
    d2 = dest.reshape(_T, _K)
    p0 = d2[:, 0].astype(jnp.int32)
    p1 = d2[:, 1].astype(jnp.int32)

    block_expert = jnp.sum(
        (jnp.arange(_NB, dtype=jnp.int32)[:, None] * _B >= pad_ends[None, :]
         ).astype(jnp.int32), axis=1)
    block_expert = jnp.minimum(block_expert, _NE - 1).astype(jnp.int32)

    lim = jnp.full((1, 1), swiglu_limit, jnp.float32)

    gw, dw = _decode_weights(gub, gus, dnb, dns)
    xg3 = _sc_dispatch_rows(flat.reshape(_T, 8, 128),
                            p0.reshape(_NW, _DNC, _DC),
                            p1.reshape(_NW, _DNC, _DC))
    r = _ffn_grouped(block_expert, lim, xg3.reshape(_P, _D), gw, b1p, dw, b2r)
    p01 = jnp.concatenate([p0, p1])
    g3 = _sc_gather_rows(r.reshape(_P, 8, 128), p01, _K * _T)
    out = _combine_add(g3.reshape(_K * _T, _D), routing_weights, ri)
    return out.reshape(*hidden_states.shape[:-1], _D).astype(hidden_states.dtype)


# speedup vs baseline: 2.3049x; 1.1392x over previous
import functools

import numpy as np
import jax
import jax.numpy as jnp
from jax import lax
from jax.experimental import pallas as pl
from jax.experimental.pallas import tpu as pltpu
from jax.experimental.pallas import tpu_sc as plsc

_NE = 8
_D = 1024
_T = 4096
_K = 2
_NP = _T * _K
_B = 256
_P = _NP + _NE * _B
_NB = _P // _B
_NW = 32


def _consts():
    p = np.arange(_D)
    b = (p >= _D // 2).astype(np.int64)
    m = p - (_D // 2) * b
    kperm = 32 * (m // 16) + 2 * (m % 16) + b
    mm = np.zeros((_D, _D), np.float32)
    mm[np.arange(_D), kperm] = 1.0
    m_lo = mm[: _D // 2]
    m_hi = mm[_D // 2:]
    s = np.zeros((32, _D // 2), np.float32)
    s[np.arange(_D // 2) // 16, np.arange(_D // 2)] = 1.0
    rowperm = np.concatenate([2 * kperm, 2 * kperm + 1])
    return m_lo, m_hi, s, rowperm


_MLO, _MHI, _SSEL, _ROWPERM = _consts()


def _decode_fp4(nib, scale):
    mag = nib & 7
    mf = mag.astype(jnp.float32)
    dec = jnp.where(mag < 5, mf * 0.5, mf - 2.0)
    dec = jnp.where(mag == 7, 6.0, dec)
    sgn = jnp.where((nib & 8) == 0, 1.0, -1.0)
    return dec * sgn * scale


def _halves(bref, sref, ssel):
    bv = bref[0].astype(jnp.int32)
    sv = sref[0].astype(jnp.int32)
    sc32 = jax.lax.bitcast_convert_type(sv << 23, jnp.float32)
    scale = jax.lax.dot_general(
        sc32, ssel, (((1,), (0,)), ((), ())),
        preferred_element_type=jnp.float32)
    lo = _decode_fp4(bv & 15, scale).astype(jnp.bfloat16)
    hi = _decode_fp4(bv >> 4, scale).astype(jnp.bfloat16)
    return lo, hi


def _decode_body(gub, gus, dnb, dns, mlo_ref, mhi_ref, ssel_ref, gw_out,
                 dw_out):
    ssel = ssel_ref[...]
    lo, hi = _halves(gub, gus, ssel)
    nat = jax.lax.dot_general(
        lo, mlo_ref[...], (((1,), (0,)), ((), ())),
        preferred_element_type=jnp.float32)
    nat = nat + jax.lax.dot_general(
        hi, mhi_ref[...], (((1,), (0,)), ((), ())),
        preferred_element_type=jnp.float32)
    gw_out[0] = nat.astype(jnp.bfloat16)
    lo, hi = _halves(dnb, dns, ssel)
    dw_out[0, :, : _D // 2] = lo
    dw_out[0, :, _D // 2:] = hi


def _decode_weights(gub, gus, dnb, dns):
    return pl.pallas_call(
        _decode_body,
        grid=(_NE,),
        in_specs=[
            pl.BlockSpec((1, 2 * _D, _D // 2), lambda e: (e, 0, 0)),
            pl.BlockSpec((1, 2 * _D, 32), lambda e: (e, 0, 0)),
            pl.BlockSpec((1, _D, _D // 2), lambda e: (e, 0, 0)),
            pl.BlockSpec((1, _D, 32), lambda e: (e, 0, 0)),
            pl.BlockSpec((_D // 2, _D), lambda e: (0, 0)),
            pl.BlockSpec((_D // 2, _D), lambda e: (0, 0)),
            pl.BlockSpec((32, _D // 2), lambda e: (0, 0)),
        ],
        out_specs=[
            pl.BlockSpec((1, 2 * _D, _D), lambda e: (e, 0, 0)),
            pl.BlockSpec((1, _D, _D), lambda e: (e, 0, 0)),
        ],
        out_shape=[
            jax.ShapeDtypeStruct((_NE, 2 * _D, _D), jnp.bfloat16),
            jax.ShapeDtypeStruct((_NE, _D, _D), jnp.bfloat16),
        ],
    )(gub, gus, dnb, dns, jnp.asarray(_MLO, jnp.bfloat16),
      jnp.asarray(_MHI, jnp.bfloat16), jnp.asarray(_SSEL, jnp.float32))


_DC = 32
_DNC = (_T // _NW) // _DC


def _sc_dispatch_rows(x3, p0r, p1r):
    per_w = _T // _NW
    mesh = plsc.VectorSubcoreMesh(core_axis_name="c", subcore_axis_name="s")

    @functools.partial(
        pl.kernel,
        mesh=mesh,
        out_type=jax.ShapeDtypeStruct((_P, _D), jnp.float32),
        scratch_types=[
            pltpu.VMEM((_DNC, _DC), jnp.int32),
            pltpu.VMEM((_DNC, _DC), jnp.int32),
            pltpu.VMEM((_DC, _D), jnp.float32),
            pltpu.VMEM((_DC, _D), jnp.float32),
            pltpu.SemaphoreType.DMA,
            pltpu.SemaphoreType.DMA,
        ],
    )
    def k(x_hbm, p0_hbm, p1_hbm, out_hbm, i0_v, i1_v, r0_v, r1_v, s0, s1):
        wid = lax.axis_index("s") * 2 + lax.axis_index("c")
        base = wid * per_w
        pltpu.sync_copy(p0_hbm.at[wid], i0_v)
        pltpu.sync_copy(p1_hbm.at[wid], i1_v)
        bufs = (r0_v, r1_v)
        sems = (s0, s1)
        hs = [None] * _DNC
        for c in range(_DNC):
            if c >= 2:
                hs[c - 2][0].wait()
                hs[c - 2][1].wait()
            pltpu.sync_copy(x_hbm.at[pl.ds(base + c * _DC, _DC)], bufs[c % 2])
            h0 = pltpu.async_copy(bufs[c % 2], out_hbm.at[i0_v.at[c]],
                                  sems[c % 2])
            h1 = pltpu.async_copy(bufs[c % 2], out_hbm.at[i1_v.at[c]],
                                  sems[c % 2])
            hs[c] = (h0, h1)
        for c in range(max(_DNC - 2, 0), _DNC):
            hs[c][0].wait()
            hs[c][1].wait()

    return k(x3, p0r, p1r)


def _sc_gather_rows(table3, rid, n_rows):
    per_w = n_rows // _NW
    gc = 32
    n_chunks = per_w // gc
    mesh = plsc.VectorSubcoreMesh(core_axis_name="c", subcore_axis_name="s")

    @functools.partial(
        pl.kernel,
        mesh=mesh,
        out_type=jax.ShapeDtypeStruct((n_rows, _D), jnp.float32),
        scratch_types=[
            pltpu.VMEM((per_w,), jnp.int32),
            pltpu.VMEM((gc, _D), jnp.float32),
            pltpu.VMEM((gc, _D), jnp.float32),
            pltpu.SemaphoreType.DMA,
            pltpu.SemaphoreType.DMA,
        ],
    )
    def k(table_hbm, rid_hbm, out_hbm, idx_v, b0, b1, s0, s1):
        wid = lax.axis_index("s") * 2 + lax.axis_index("c")
        base = wid * per_w
        pltpu.sync_copy(rid_hbm.at[pl.ds(base, per_w)], idx_v)
        bufs = (b0, b1)
        sems = (s0, s1)
        hs = [None] * n_chunks
        hs[0] = pltpu.async_copy(
            table_hbm.at[idx_v.at[pl.ds(0, gc)]], bufs[0], sems[0])
        for c in range(n_chunks):
            if c + 1 < n_chunks:
                hs[c + 1] = pltpu.async_copy(
                    table_hbm.at[idx_v.at[pl.ds((c + 1) * gc, gc)]],
                    bufs[(c + 1) % 2], sems[(c + 1) % 2])
            hs[c].wait()
            pltpu.sync_copy(bufs[c % 2], out_hbm.at[pl.ds(base + c * gc, gc)])

    return k(table3, rid)


def _ffn_body(be_ref, lim_ref, x_ref, gw_ref, b1_ref, dw_ref, b2_ref, out_ref):
    limv = lim_ref[0, 0]
    x = x_ref[...].astype(jnp.bfloat16)
    gu = jax.lax.dot_general(
        x, gw_ref[0], (((1,), (1,)), ((), ())),
        preferred_element_type=jnp.float32)
    gu = gu + b1_ref[0]
    g = jnp.minimum(gu[:, :_D], limv)
    l = jnp.clip(gu[:, _D:], -limv, limv)
    act = (g * jax.nn.sigmoid(1.702 * g) * (l + 1.0)).astype(jnp.bfloat16)
    y = jax.lax.dot_general(
        act, dw_ref[0], (((1,), (1,)), ((), ())),
        preferred_element_type=jnp.float32)
    y = y + b2_ref[0]
    out_ref[...] = y


def _ffn_grouped(block_expert, lim, xg, gw, b1, dw, b2):
    grid_spec = pltpu.PrefetchScalarGridSpec(
        num_scalar_prefetch=1,
        grid=(_NB,),
        in_specs=[
            pl.BlockSpec(memory_space=pltpu.SMEM),
            pl.BlockSpec((_B, _D), lambda i, be: (i, 0)),
            pl.BlockSpec((1, 2 * _D, _D), lambda i, be: (be[i], 0, 0)),
            pl.BlockSpec((1, 1, 2 * _D), lambda i, be: (be[i], 0, 0)),
            pl.BlockSpec((1, _D, _D), lambda i, be: (be[i], 0, 0)),
            pl.BlockSpec((1, 1, _D), lambda i, be: (be[i], 0, 0)),
        ],
        out_specs=pl.BlockSpec((_B, _D), lambda i, be: (i, 0)),
    )
    return pl.pallas_call(
        _ffn_body,
        grid_spec=grid_spec,
        out_shape=jax.ShapeDtypeStruct((_P, _D), jnp.float32),
    )(block_expert, lim, xg, gw, b1, dw, b2)


def _combine_body(a_ref, b_ref, rw_ref, ri_ref, out_ref):
    ri = ri_ref[...]
    rw = rw_ref[...]
    eids = jax.lax.broadcasted_iota(jnp.int32, (ri.shape[0], _NE), 1)
    w0 = jnp.sum(jnp.where(eids == ri[:, 0:1], rw, 0.0), axis=1)
    w1 = jnp.sum(jnp.where(eids == ri[:, 1:2], rw, 0.0), axis=1)
    out_ref[...] = w0[:, None] * a_ref[...] + w1[:, None] * b_ref[...]


def _combine_add(g, rw, ri):
    nb = 8
    rb = _T // nb
    return pl.pallas_call(
        _combine_body,
        grid=(nb,),
        in_specs=[
            pl.BlockSpec((rb, _D), lambda i: (i, 0)),
            pl.BlockSpec((rb, _D), lambda i: (i + nb, 0)),
            pl.BlockSpec((rb, _NE), lambda i: (i, 0)),
            pl.BlockSpec((rb, _K), lambda i: (i, 0)),
        ],
        out_specs=pl.BlockSpec((rb, _D), lambda i: (i, 0)),
        out_shape=jax.ShapeDtypeStruct((_T, _D), jnp.float32),
    )(g, g, rw, ri)


def kernel(hidden_states, router_indices, routing_weights, swiglu_limit,
           gate_up_proj_blocks, gate_up_proj_scales, gate_up_proj_bias,
           down_proj_blocks, down_proj_scales, down_proj_bias):
    flat = hidden_states.reshape(-1, _D)

    def rowperm(a, tail):
        return a.reshape(_NE, 32, 16, 2, 2, *tail).transpose(
            0, 4, 3, 1, 2, *range(5, 5 + len(tail))).reshape(
            _NE, 2 * _D, *tail)

    gub = rowperm(gate_up_proj_blocks.reshape(_NE, 2 * _D, _D // 2), (_D // 2,))
    gus = rowperm(gate_up_proj_scales, (32,))
    dnb = down_proj_blocks.reshape(_NE, _D, _D // 2)
    dns = down_proj_scales
    b1p = rowperm(gate_up_proj_bias, ()).reshape(_NE, 1, 2 * _D)
    b2r = down_proj_bias.reshape(_NE, 1, _D)

    ri = router_indices.astype(jnp.int32)
    e_flat = ri.reshape(-1)
    onehot = (e_flat[:, None] == jnp.arange(_NE, dtype=jnp.int32)[None, :]
              ).astype(jnp.int32)
    ranks_inc = jnp.cumsum(onehot, axis=0)
    counts = ranks_inc[-1]
    padded = ((counts + _B - 1) // _B) * _B
    pad_ends = jnp.cumsum(padded)
    pad_offs = pad_ends - padded
    rank = jnp.sum((ranks_inc - onehot) * onehot, axis=1)
    dest = jnp.sum(pad_offs[None, :] * onehot, axis=1) + rank
    d2 = dest.reshape(_T, _K)
    p0 = d2[:, 0].astype(jnp.int32)
    p1 = d2[:, 1].astype(jnp.int32)

    block_expert = jnp.sum(
        (jnp.arange(_NB, dtype=jnp.int32)[:, None] * _B >= pad_ends[None, :]
         ).astype(jnp.int32), axis=1)
    block_expert = jnp.minimum(block_expert, _NE - 1).astype(jnp.int32)

    lim = jnp.full((1, 1), swiglu_limit, jnp.float32)

    gw, dw = _decode_weights(gub, gus, dnb, dns)
    xg = _sc_dispatch_rows(flat,
                           p0.reshape(_NW, _DNC, _DC),
                           p1.reshape(_NW, _DNC, _DC))
    r = _ffn_grouped(block_expert, lim, xg, gw, b1p, dw, b2r)
    p01 = jnp.concatenate([p0, p1])
    g = _sc_gather_rows(r, p01, _K * _T)
    out = _combine_add(g, routing_weights, ri)
    return out.reshape(*hidden_states.shape[:-1], _D).astype(hidden_states.dtype)
